# baseline (speedup 1.0000x reference)
import jax
import jax.numpy as jnp
from jax.experimental import pallas as pl
from jax.experimental.pallas import tpu as pltpu
from jax.experimental.pallas import tpu_sc as plsc

_WINDOW = 256


def kernel(input_ids, table):
    batch, fields = input_ids.shape
    emb, dim = table.shape
    num_idx = batch * fields
    assert num_idx % _WINDOW == 0
    grid = num_idx // _WINDOW

    table2 = table.reshape(emb // 2, 2 * dim)
    idx_flat = input_ids.reshape(1, num_idx)
    idx_pair = idx_flat >> 1

    mesh = plsc.VectorSubcoreMesh(
        core_axis_name="core", subcore_axis_name="subcore"
    )

    @pl.kernel(
        out_type=jax.ShapeDtypeStruct((num_idx, 2 * dim), table.dtype),
        mesh=mesh,
    )
    def gather_kernel(table_hbm, idx_hbm, out_hbm):
        def body(idx_vmem, out_vmem):
            pltpu.sync_copy(table_hbm.at[idx_vmem.at[0]], out_vmem)

        pltpu.emit_pipeline(
            body,
            grid=(grid,),
            in_specs=[
                pl.BlockSpec((1, _WINDOW), index_map=lambda i: (0, i))
            ],
            out_specs=[
                pl.BlockSpec(
                    (_WINDOW, 2 * dim), index_map=lambda i: (i, 0)
                )
            ],
            core_axis_name=("core", "subcore"),
            dimension_semantics=(pltpu.PARALLEL,),
        )(idx_hbm, out_hbm)

    pairs = gather_kernel(table2, idx_pair)

    halves = pairs.reshape(2 * num_idx, dim)
    idx_half = 2 * jnp.arange(num_idx, dtype=jnp.int32) + (idx_flat[0] & 1)
    out = jnp.take(halves, idx_half, axis=0, mode="clip")
    return out.reshape(batch, fields, dim)


# speedup vs baseline: 1.1852x; 1.1852x over previous
import jax
import jax.numpy as jnp
from jax.experimental import pallas as pl
from jax.experimental.pallas import tpu as pltpu
from jax.experimental.pallas import tpu_sc as plsc

_WINDOW = 256


def kernel(input_ids, table):
    batch, fields = input_ids.shape
    emb, dim = table.shape
    num_idx = batch * fields
    assert num_idx % _WINDOW == 0
    grid = num_idx // _WINDOW

    idx_fm = input_ids.T.reshape(grid, 1, _WINDOW)
    idx_pair = idx_fm >> 1

    table2 = table.reshape(emb // 2, 2 * dim)

    mesh = plsc.VectorSubcoreMesh(
        core_axis_name="core", subcore_axis_name="subcore"
    )

    @pl.kernel(
        out_type=jax.ShapeDtypeStruct((num_idx, 2 * dim), table.dtype),
        mesh=mesh,
    )
    def gather_kernel(table_hbm, idx_hbm, out_hbm):
        def body(idx_vmem, out_vmem):
            pltpu.sync_copy(table_hbm.at[idx_vmem.at[0, 0]], out_vmem)

        pltpu.emit_pipeline(
            body,
            grid=(grid,),
            in_specs=[
                pl.BlockSpec((1, 1, _WINDOW), index_map=lambda i: (i, 0, 0))
            ],
            out_specs=[
                pl.BlockSpec(
                    (_WINDOW, 2 * dim), index_map=lambda i: (i, 0)
                )
            ],
            core_axis_name=("core", "subcore"),
            dimension_semantics=(pltpu.PARALLEL,),
        )(idx_hbm, out_hbm)

    pairs = gather_kernel(table2, idx_pair)

    halves = pairs.reshape(2 * num_idx, dim)
    parity = (idx_fm & 1).reshape(num_idx)
    idx_half = 2 * jnp.arange(num_idx, dtype=jnp.int32) + parity
    out_fm = jnp.take(halves, idx_half, axis=0, mode="clip")

    return jnp.transpose(out_fm.reshape(fields, batch, dim), (1, 0, 2))
